# baseline (speedup 1.0000x reference)
import functools

import jax
import jax.numpy as jnp
from jax import lax
from jax.experimental import pallas as pl
from jax.experimental.pallas import tpu as pltpu
from jax.experimental.pallas import tpu_sc as plsc

B = 4096
LT = 50
LG = 20
D = 32
NC = 2
NS = 16
NW = NC * NS
BPW = B // NW
NBUF = 8

_MESH = plsc.VectorSubcoreMesh(
    core_axis_name="c", subcore_axis_name="s", num_cores=NC, num_subcores=NS
)


def _sc_idemb_body(uT_hbm, mT_hbm, uid_hbm, mid_hbm, u_out, m_out,
                   idx_v, rows_v, *ring):
    bufs, sems = ring[:NBUF], ring[NBUF:]
    wid = lax.axis_index("s") * NC + lax.axis_index("c")
    base = wid * BPW
    lane = lax.iota(jnp.int32, 16)

    def gather_table(tabT_hbm, ids_hbm, out_hbm):
        pltpu.sync_copy(ids_hbm.at[pl.ds(base, BPW)], idx_v)

        def jof(i):
            c = idx_v[pl.ds(pl.multiple_of((i >> 4) << 4, 16), 16)]
            return jnp.sum(jnp.where(lane == (i & 15), c, 0))

        def fetch(i, buf, sem):
            j = jof(jnp.minimum(i, BPW - 1))
            jb = pl.multiple_of((j >> 7) << 7, 128)
            pltpu.async_copy(tabT_hbm.at[:, pl.ds(jb, 128)], buf, sem)

        def extract(i, buf):
            j = jof(i)
            jm = jnp.broadcast_to(j & 127, (16,))
            rows_v[i, pl.ds(0, 16)] = plsc.load_gather(buf, [lane, jm])
            rows_v[i, pl.ds(16, 16)] = plsc.load_gather(buf, [lane + 16, jm])

        for b in range(NBUF):
            fetch(b, bufs[b], sems[b])

        @pl.loop(0, BPW, step=NBUF)
        def _blk(i):
            for b in range(NBUF):
                pltpu.make_async_copy(
                    tabT_hbm.at[:, pl.ds(0, 128)], bufs[b], sems[b]).wait()
                extract(i + b, bufs[b])
                fetch(i + b + NBUF, bufs[b], sems[b])

        for b in range(NBUF):
            pltpu.make_async_copy(
                tabT_hbm.at[:, pl.ds(0, 128)], bufs[b], sems[b]).wait()

        pltpu.sync_copy(rows_v, out_hbm.at[pl.ds(base, BPW)])

    gather_table(uT_hbm, uid_hbm, u_out)
    gather_table(mT_hbm, mid_hbm, m_out)


@jax.jit
def _sc_idemb(uT, mT, user_id, movie_id):
    f32 = jnp.float32
    return pl.kernel(
        _sc_idemb_body,
        out_type=[
            jax.ShapeDtypeStruct((B, D), f32),
            jax.ShapeDtypeStruct((B, D), f32),
        ],
        mesh=_MESH,
        compiler_params=pltpu.CompilerParams(
            use_tc_tiling_on_sc=True, needs_layout_passes=False),
        scratch_types=[
            pltpu.VMEM((BPW,), jnp.int32),
            pltpu.VMEM((BPW, D), f32),
        ] + [pltpu.VMEM((D, 128), f32) for _ in range(NBUF)]
          + [pltpu.SemaphoreType.DMA for _ in range(NBUF)],
    )(uT, mT, user_id, movie_id)


def _transpose_body(xT_ref, out_ref):
    x = xT_ref[...]
    y = jnp.transpose(x)
    blk = y.shape[0]
    g = 128 // D
    y3 = jnp.reshape(y, (blk // g, g, D))
    cat = jnp.concatenate([y3[:, c, :] for c in range(g)], axis=1)
    out_ref[...] = jnp.reshape(cat, (blk * D,))


def _make_row_major(tableT, rows, cols, blk):
    n = -(-rows // blk)
    flat = pl.pallas_call(
        _transpose_body,
        grid=(n,),
        in_specs=[pl.BlockSpec((cols, blk), lambda i: (0, i))],
        out_specs=pl.BlockSpec((blk * cols,), lambda i: (i,)),
        out_shape=jax.ShapeDtypeStruct((rows * cols,), jnp.float32),
    )(tableT)
    return flat.reshape(rows, cols)


NPB = 5


def _sc_pool_body(tagT_hbm, genT_hbm, tag_tab, gen_tab,
                  tsum_out, gsum_out,
                  acc_v, r0, r1, r2, r3, r4, tidx_v, gidx_v,
                  sa, s0, s1, s2, s3, s4):
    wid = lax.axis_index("s") * NC + lax.axis_index("c")
    base = wid * BPW
    bufs = (r0, r1, r2, r3, r4)
    sems = (s0, s1, s2, s3, s4)

    def pooled_sum(idsT_hbm, L, table, out_hbm, lidx_v):
        pltpu.sync_copy(idsT_hbm.at[:, pl.ds(base, BPW)], lidx_v)

        def fetch(l, buf, sem):
            pltpu.async_copy(
                table.at[lidx_v.at[jnp.minimum(l, L - 1)]], buf, sem)

        def wait(buf, sem):
            pltpu.make_async_copy(table.at[lidx_v.at[0]], buf, sem).wait()

        def accum(buf):
            @pl.loop(0, BPW, unroll=8)
            def _acc(r):
                plsc.addupdate(acc_v.at[r, pl.ds(0, 16)], buf[r, pl.ds(0, 16)])
                plsc.addupdate(acc_v.at[r, pl.ds(16, 16)], buf[r, pl.ds(16, 16)])

        P = (L - 1) // NPB

        da = pltpu.async_copy(table.at[lidx_v.at[0]], acc_v, sa)
        for b in range(NPB):
            fetch(1 + b, bufs[b], sems[b])
        da.wait()

        @pl.loop(0, P)
        def _round(t):
            for b in range(NPB):
                l = NPB * t + 1 + b
                wait(bufs[b], sems[b])
                accum(bufs[b])
                fetch(l + NPB, bufs[b], sems[b])

        for b in range(4):
            wait(bufs[b], sems[b])
            accum(bufs[b])
        wait(bufs[4], sems[4])

        pltpu.sync_copy(acc_v, out_hbm.at[pl.ds(base, BPW)])

    pooled_sum(tagT_hbm, LT, tag_tab, tsum_out, tidx_v)
    pooled_sum(genT_hbm, LG, gen_tab, gsum_out, gidx_v)


@jax.jit
def _sc_pool(tagT, genT, tag_table, genre_table):
    f32 = jnp.float32
    return pl.kernel(
        _sc_pool_body,
        out_type=[
            jax.ShapeDtypeStruct((B, D), f32),
            jax.ShapeDtypeStruct((B, D), f32),
        ],
        mesh=_MESH,
        compiler_params=pltpu.CompilerParams(
            use_tc_tiling_on_sc=False, needs_layout_passes=False),
        scratch_types=[pltpu.VMEM((BPW, D), f32) for _ in range(6)] + [
            pltpu.VMEM((LT, BPW), jnp.int32),
            pltpu.VMEM((LG, BPW), jnp.int32),
        ] + [pltpu.SemaphoreType.DMA for _ in range(6)],
    )(tagT, genT, tag_table, genre_table)


def _tc_body(uemb_ref, tsum_ref, tagT_ref, iemb_ref, gsum_ref, genT_ref,
             trow0_ref, grow0_ref,
             uW1_ref, uB1_ref, uW2_ref, uB2_ref,
             iW1_ref, iB1_ref, iW2_ref, iB2_ref, out_ref):
    def pooled(sum_ref, idsT_ref, row0_ref, L):
        ids = idsT_ref[...]
        nnz = jnp.sum((ids != 0).astype(jnp.float32), axis=0).reshape(B, 1)
        fallback = jnp.min(nnz) == 0.0
        denom = jnp.where(fallback, float(L), jnp.maximum(nnz, 1.0))
        eff = jnp.where(fallback, sum_ref[...],
                        sum_ref[...] - (float(L) - nnz) * row0_ref[...])
        return eff / denom

    tpool = pooled(tsum_ref, tagT_ref, trow0_ref, LT)
    gpool = pooled(gsum_ref, genT_ref, grow0_ref, LG)

    def mlp(x, W1, b1, W2, b2):
        h = jnp.maximum(
            jnp.dot(x, W1, preferred_element_type=jnp.float32) + b1, 0.0)
        return jnp.dot(h, W2, preferred_element_type=jnp.float32) + b2

    user_in = jnp.concatenate([uemb_ref[...], tpool], axis=1)
    item_in = jnp.concatenate([iemb_ref[...], gpool], axis=1)
    uvec = mlp(user_in, uW1_ref[...], uB1_ref[...], uW2_ref[...], uB2_ref[...])
    ivec = mlp(item_in, iW1_ref[...], iB1_ref[...], iW2_ref[...], iB2_ref[...])
    out_ref[...] = jnp.sum(uvec * ivec, axis=1)


@jax.jit
def _tc_towers(uemb, tsum, tagT, iemb, gsum, genT, trow0, grow0,
               uW1, uB1, uW2, uB2, iW1, iB1, iW2, iB2):
    return pl.pallas_call(
        _tc_body,
        out_shape=jax.ShapeDtypeStruct((B,), jnp.float32),
    )(uemb, tsum, tagT, iemb, gsum, genT, trow0, grow0,
      uW1, uB1, uW2, uB2, iW1, iB1, iW2, iB2)


def kernel(user_id, tag_input_ids, movie_id, genre_input_ids,
           user_table, movie_table, tag_table, genre_table,
           uW1, uB1, uW2, uB2, iW1, iB1, iW2, iB2):
    user_id = user_id.astype(jnp.int32)
    movie_id = movie_id.astype(jnp.int32)
    tagT = tag_input_ids.astype(jnp.int32).T
    genT = genre_input_ids.astype(jnp.int32).T

    uemb, iemb = _sc_idemb(user_table.T, movie_table.T, user_id, movie_id)
    tag_lin = tag_table.reshape(25000, 128).reshape(100000, D)
    gen_lin = genre_table.reshape(250, 128).reshape(1000, D)
    tag_lin_b, gen_lin_b, _, _ = lax.optimization_barrier(
        (tag_lin, gen_lin, uemb, iemb))
    tsum, gsum = _sc_pool(tagT, genT, tag_lin_b, gen_lin_b)

    return _tc_towers(
        uemb, tsum, tagT, iemb, gsum, genT,
        tag_table[0:1], genre_table[0:1],
        uW1, uB1.reshape(1, -1), uW2, uB2.reshape(1, -1),
        iW1, iB1.reshape(1, -1), iW2, iB2.reshape(1, -1))


# speedup vs baseline: 1.2232x; 1.2232x over previous
import functools

import jax
import jax.numpy as jnp
from jax import lax
from jax.experimental import pallas as pl
from jax.experimental.pallas import tpu as pltpu
from jax.experimental.pallas import tpu_sc as plsc

B = 4096
LT = 50
LG = 20
D = 32
NC = 2
NS = 16
NW = NC * NS
BPW = B // NW
NBUF = 8

_MESH = plsc.VectorSubcoreMesh(
    core_axis_name="c", subcore_axis_name="s", num_cores=NC, num_subcores=NS
)


def _sc_idemb_body(uT_hbm, mT_hbm, uid_hbm, mid_hbm, u_out, m_out,
                   idx_v, rows_v, *ring):
    bufs, sems = ring[:NBUF], ring[NBUF:]
    wid = lax.axis_index("s") * NC + lax.axis_index("c")
    base = wid * BPW
    lane = lax.iota(jnp.int32, 16)

    def gather_table(tabT_hbm, ids_hbm, out_hbm):
        pltpu.sync_copy(ids_hbm.at[pl.ds(base, BPW)], idx_v)

        def jof(i):
            c = idx_v[pl.ds(pl.multiple_of((i >> 4) << 4, 16), 16)]
            return jnp.sum(jnp.where(lane == (i & 15), c, 0))

        def fetch(i, buf, sem):
            j = jof(jnp.minimum(i, BPW - 1))
            jb = pl.multiple_of((j >> 7) << 7, 128)
            pltpu.async_copy(tabT_hbm.at[:, pl.ds(jb, 128)], buf, sem)

        def extract(i, buf):
            j = jof(i)
            jm = jnp.broadcast_to(j & 127, (16,))
            rows_v[i, pl.ds(0, 16)] = plsc.load_gather(buf, [lane, jm])
            rows_v[i, pl.ds(16, 16)] = plsc.load_gather(buf, [lane + 16, jm])

        for b in range(NBUF):
            fetch(b, bufs[b], sems[b])

        @pl.loop(0, BPW, step=NBUF)
        def _blk(i):
            for b in range(NBUF):
                pltpu.make_async_copy(
                    tabT_hbm.at[:, pl.ds(0, 128)], bufs[b], sems[b]).wait()
                extract(i + b, bufs[b])
                fetch(i + b + NBUF, bufs[b], sems[b])

        for b in range(NBUF):
            pltpu.make_async_copy(
                tabT_hbm.at[:, pl.ds(0, 128)], bufs[b], sems[b]).wait()

        pltpu.sync_copy(rows_v, out_hbm.at[pl.ds(base, BPW)])

    gather_table(uT_hbm, uid_hbm, u_out)
    gather_table(mT_hbm, mid_hbm, m_out)


@jax.jit
def _sc_idemb(uT, mT, user_id, movie_id):
    f32 = jnp.float32
    return pl.kernel(
        _sc_idemb_body,
        out_type=[
            jax.ShapeDtypeStruct((B, D), f32),
            jax.ShapeDtypeStruct((B, D), f32),
        ],
        mesh=_MESH,
        compiler_params=pltpu.CompilerParams(
            use_tc_tiling_on_sc=True, needs_layout_passes=False),
        scratch_types=[
            pltpu.VMEM((BPW,), jnp.int32),
            pltpu.VMEM((BPW, D), f32),
        ] + [pltpu.VMEM((D, 128), f32) for _ in range(NBUF)]
          + [pltpu.SemaphoreType.DMA for _ in range(NBUF)],
    )(uT, mT, user_id, movie_id)


def _transpose_body(xT_ref, out_ref):
    x = xT_ref[...]
    y = jnp.transpose(x)
    blk = y.shape[0]
    g = 128 // D
    y3 = jnp.reshape(y, (blk // g, g, D))
    cat = jnp.concatenate([y3[:, c, :] for c in range(g)], axis=1)
    out_ref[...] = jnp.reshape(cat, (blk * D,))


def _make_row_major(tableT, rows, cols, blk):
    n = -(-rows // blk)
    flat = pl.pallas_call(
        _transpose_body,
        grid=(n,),
        in_specs=[pl.BlockSpec((cols, blk), lambda i: (0, i))],
        out_specs=pl.BlockSpec((blk * cols,), lambda i: (i,)),
        out_shape=jax.ShapeDtypeStruct((rows * cols,), jnp.float32),
    )(tableT)
    return flat.reshape(rows, cols)


NPB = 5


def _sc_pool_body(tagT_hbm, genT_hbm, tag_tab, gen_tab,
                  tsum_out, gsum_out,
                  acc_v, r0, r1, r2, r3, r4, tidx_v, gidx_v,
                  sa, s0, s1, s2, s3, s4):
    wid = lax.axis_index("s") * NC + lax.axis_index("c")
    base = wid * BPW
    bufs = (r0, r1, r2, r3, r4)
    sems = (s0, s1, s2, s3, s4)

    def pooled_sum(idsT_hbm, L, table, out_hbm, lidx_v):
        pltpu.sync_copy(idsT_hbm.at[:, pl.ds(base, BPW)], lidx_v)

        def fetch(l, buf, sem):
            pltpu.async_copy(
                table.at[lidx_v.at[jnp.minimum(l, L - 1)]], buf, sem)

        def wait(buf, sem):
            pltpu.make_async_copy(table.at[lidx_v.at[0]], buf, sem).wait()

        def accum(buf):
            @pl.loop(0, BPW, unroll=8)
            def _acc(r):
                plsc.addupdate(acc_v.at[r, pl.ds(0, 16)], buf[r, pl.ds(0, 16)])
                plsc.addupdate(acc_v.at[r, pl.ds(16, 16)], buf[r, pl.ds(16, 16)])

        P = (L - 1) // NPB

        da = pltpu.async_copy(table.at[lidx_v.at[0]], acc_v, sa)
        for b in range(NPB):
            fetch(1 + b, bufs[b], sems[b])
        da.wait()

        @pl.loop(0, P)
        def _round(t):
            for b in range(NPB):
                l = NPB * t + 1 + b
                wait(bufs[b], sems[b])
                accum(bufs[b])
                fetch(l + NPB, bufs[b], sems[b])

        for b in range(4):
            wait(bufs[b], sems[b])
            accum(bufs[b])
        wait(bufs[4], sems[4])

        pltpu.sync_copy(acc_v, out_hbm.at[pl.ds(base, BPW)])

    pooled_sum(tagT_hbm, LT, tag_tab, tsum_out, tidx_v)
    pooled_sum(genT_hbm, LG, gen_tab, gsum_out, gidx_v)


@jax.jit
def _sc_pool(tagT, genT, tag_table, genre_table):
    f32 = jnp.float32
    return pl.kernel(
        _sc_pool_body,
        out_type=[
            jax.ShapeDtypeStruct((B, D), f32),
            jax.ShapeDtypeStruct((B, D), f32),
        ],
        mesh=_MESH,
        compiler_params=pltpu.CompilerParams(
            use_tc_tiling_on_sc=False, needs_layout_passes=False),
        scratch_types=[pltpu.VMEM((BPW, D), f32) for _ in range(6)] + [
            pltpu.VMEM((LT, BPW), jnp.int32),
            pltpu.VMEM((LG, BPW), jnp.int32),
        ] + [pltpu.SemaphoreType.DMA for _ in range(6)],
    )(tagT, genT, tag_table, genre_table)


def _tc_body(uemb_ref, tsum_ref, tagT_ref, iemb_ref, gsum_ref, genT_ref,
             trow0_ref, grow0_ref,
             uW1_ref, uB1_ref, uW2_ref, uB2_ref,
             iW1_ref, iB1_ref, iW2_ref, iB2_ref, out_ref):
    def pooled(sum_ref, idsT_ref, row0_ref, L):
        ids = idsT_ref[...]
        nnz = jnp.sum((ids != 0).astype(jnp.float32), axis=0).reshape(B, 1)
        fallback = jnp.min(nnz) == 0.0
        denom = jnp.where(fallback, float(L), jnp.maximum(nnz, 1.0))
        eff = jnp.where(fallback, sum_ref[...],
                        sum_ref[...] - (float(L) - nnz) * row0_ref[...])
        return eff / denom

    tpool = pooled(tsum_ref, tagT_ref, trow0_ref, LT)
    gpool = pooled(gsum_ref, genT_ref, grow0_ref, LG)

    def mlp(x, W1, b1, W2, b2):
        h = jnp.maximum(
            jnp.dot(x, W1, preferred_element_type=jnp.float32) + b1, 0.0)
        return jnp.dot(h, W2, preferred_element_type=jnp.float32) + b2

    user_in = jnp.concatenate([uemb_ref[...], tpool], axis=1)
    item_in = jnp.concatenate([iemb_ref[...], gpool], axis=1)
    uvec = mlp(user_in, uW1_ref[...], uB1_ref[...], uW2_ref[...], uB2_ref[...])
    ivec = mlp(item_in, iW1_ref[...], iB1_ref[...], iW2_ref[...], iB2_ref[...])
    out_ref[...] = jnp.sum(uvec * ivec, axis=1)


@jax.jit
def _tc_towers(uemb, tsum, tagT, iemb, gsum, genT, trow0, grow0,
               uW1, uB1, uW2, uB2, iW1, iB1, iW2, iB2):
    return pl.pallas_call(
        _tc_body,
        out_shape=jax.ShapeDtypeStruct((B,), jnp.float32),
    )(uemb, tsum, tagT, iemb, gsum, genT, trow0, grow0,
      uW1, uB1, uW2, uB2, iW1, iB1, iW2, iB2)


def kernel(user_id, tag_input_ids, movie_id, genre_input_ids,
           user_table, movie_table, tag_table, genre_table,
           uW1, uB1, uW2, uB2, iW1, iB1, iW2, iB2):
    user_id = user_id.astype(jnp.int32)
    movie_id = movie_id.astype(jnp.int32)
    tagT = tag_input_ids.astype(jnp.int32).T
    genT = genre_input_ids.astype(jnp.int32).T

    uemb, iemb = _sc_idemb(user_table.T, movie_table.T, user_id, movie_id)
    tagT_b, genT_b, uemb, iemb = lax.optimization_barrier(
        (tagT, genT, uemb, iemb))
    tag_lin = tag_table.reshape(25000, 128).reshape(100000, D)
    gen_lin = genre_table.reshape(250, 128).reshape(1000, D)
    tsum, gsum = _sc_pool(tagT_b, genT_b, tag_lin, gen_lin)

    return _tc_towers(
        uemb, tsum, tagT, iemb, gsum, genT,
        tag_table[0:1], genre_table[0:1],
        uW1, uB1.reshape(1, -1), uW2, uB2.reshape(1, -1),
        iW1, iB1.reshape(1, -1), iW2, iB2.reshape(1, -1))
